# initial kernel scaffold (unmeasured)
import jax
import jax.numpy as jnp
from jax import lax
from jax.experimental import pallas as pl
from jax.experimental.pallas import tpu as pltpu

B = 4
S = 1024
S_HALF = 512
K = 2048
N = 4096


def kernel(O, Wo):
    O2 = O.reshape(B, S, K)

    def body(o_ref, wo_ref, out_ref, send_buf, send_sems, recv_sems):
        my_x = lax.axis_index("x")
        my_y = lax.axis_index("y")
        peer = (my_x, 1 - my_y)
        my_lo = my_y * S_HALF
        peer_lo = (1 - my_y) * S_HALF

        barrier = pltpu.get_barrier_semaphore()
        pl.semaphore_signal(
            barrier, inc=1, device_id=peer, device_id_type=pl.DeviceIdType.MESH
        )
        pl.semaphore_wait(barrier, 1)

        sends = []
        for b in range(B):
            slot = b % 2
            if b >= 2:
                sends[b - 2].wait_send()
            part = jnp.dot(
                o_ref[b, pl.ds(peer_lo, S_HALF), :],
                wo_ref[...],
                preferred_element_type=jnp.float32,
            )
            send_buf[slot] = part
            rdma = pltpu.make_async_remote_copy(
                src_ref=send_buf.at[slot],
                dst_ref=out_ref.at[b],
                send_sem=send_sems.at[slot],
                recv_sem=recv_sems.at[b],
                device_id=peer,
                device_id_type=pl.DeviceIdType.MESH,
            )
            rdma.start()
            sends.append(rdma)

        for b in range(B):
            part = jnp.dot(
                o_ref[b, pl.ds(my_lo, S_HALF), :],
                wo_ref[...],
                preferred_element_type=jnp.float32,
            )
            recv = pltpu.make_async_remote_copy(
                src_ref=send_buf.at[b % 2],
                dst_ref=out_ref.at[b],
                send_sem=send_sems.at[b % 2],
                recv_sem=recv_sems.at[b],
                device_id=peer,
                device_id_type=pl.DeviceIdType.MESH,
            )
            recv.wait_recv()
            out_ref[b] = out_ref[b] + part

        sends[2].wait_send()
        sends[3].wait_send()

    return pl.pallas_call(
        body,
        out_shape=jax.ShapeDtypeStruct((B, S_HALF, N), jnp.float32),
        in_specs=[
            pl.BlockSpec(memory_space=pltpu.VMEM),
            pl.BlockSpec(memory_space=pltpu.VMEM),
        ],
        out_specs=pl.BlockSpec(memory_space=pltpu.VMEM),
        scratch_shapes=[
            pltpu.VMEM((2, S_HALF, N), jnp.float32),
            pltpu.SemaphoreType.DMA((2,)),
            pltpu.SemaphoreType.DMA((4,)),
        ],
        compiler_params=pltpu.CompilerParams(collective_id=0),
    )(O2, Wo)


# baseline (device time: 443160 ns/iter reference)
import jax

jax.config.update("jax_compilation_cache_dir", "/tmp/jax_cache")
jax.config.update("jax_persistent_cache_min_entry_size_bytes", -1)
jax.config.update("jax_persistent_cache_min_compile_time_secs", 0)

import jax.numpy as jnp
from jax import lax
from jax.experimental import pallas as pl
from jax.experimental.pallas import tpu as pltpu

B = 4
S = 1024
S_HALF = 512
R = 256
K = 2048
N = 4096
N_CHUNK = B * (S_HALF // R)


def kernel(O, Wo):
    O2 = O.reshape(B, S, K)

    def body(o_ref, wo_ref, out_ref, o_buf, s_buf, a_buf,
             copy_sem, store_sems, send_sems, recv_sems):
        my_x = lax.axis_index("x")
        my_y = lax.axis_index("y")
        peer = (my_x, 1 - my_y)
        my_lo = my_y * S_HALF
        peer_lo = (1 - my_y) * S_HALF

        barrier = pltpu.get_barrier_semaphore()
        pl.semaphore_signal(
            barrier, inc=1, device_id=peer, device_id_type=pl.DeviceIdType.MESH
        )
        pl.semaphore_wait(barrier, 1)

        def remote_desc(slot, b, j, c):
            return pltpu.make_async_remote_copy(
                src_ref=s_buf.at[slot],
                dst_ref=out_ref.at[b, pl.ds(j * R, R), :],
                send_sem=send_sems.at[slot],
                recv_sem=recv_sems.at[c],
                device_id=peer,
                device_id_type=pl.DeviceIdType.MESH,
            )

        def store_desc(slot, b, j):
            return pltpu.make_async_copy(
                a_buf.at[slot],
                out_ref.at[b, pl.ds(j * R, R), :],
                store_sems.at[slot],
            )

        def load_chunk(b, row_lo, slot):
            cp = pltpu.make_async_copy(
                o_ref.at[b, pl.ds(row_lo, R), :], o_buf.at[slot], copy_sem
            )
            cp.start()
            cp.wait()

        def phase1(c, carry):
            b, j = c // 2, c % 2
            slot = c % 2
            load_chunk(b, peer_lo + j * R, slot)

            @pl.when(c >= 2)
            def _():
                remote_desc(slot, b, j, c).wait_send()

            s_buf[slot] = jnp.dot(
                o_buf[slot], wo_ref[...], preferred_element_type=jnp.float32
            )
            remote_desc(slot, b, j, c).start()
            return carry

        lax.fori_loop(0, N_CHUNK, phase1, 0)

        def phase2(c, carry):
            b, j = c // 2, c % 2
            slot = c % 2
            load_chunk(b, my_lo + j * R, slot)
            part = jnp.dot(
                o_buf[slot], wo_ref[...], preferred_element_type=jnp.float32
            )
            remote_desc(slot, b, j, c).wait_recv()

            @pl.when(c >= 2)
            def _():
                store_desc(slot, b, j).wait()

            ld = pltpu.make_async_copy(
                out_ref.at[b, pl.ds(j * R, R), :], a_buf.at[slot], copy_sem
            )
            ld.start()
            ld.wait()
            a_buf[slot] = a_buf[slot] + part
            store_desc(slot, b, j).start()
            return carry

        lax.fori_loop(0, N_CHUNK, phase2, 0)

        for slot in range(2):
            remote_desc(slot, 0, 0, slot).wait_send()
            store_desc(slot, 0, 0).wait()

    return pl.pallas_call(
        body,
        out_shape=jax.ShapeDtypeStruct((B, S_HALF, N), jnp.float32),
        in_specs=[
            pl.BlockSpec(memory_space=pl.ANY),
            pl.BlockSpec(memory_space=pltpu.VMEM),
        ],
        out_specs=pl.BlockSpec(memory_space=pl.ANY),
        scratch_shapes=[
            pltpu.VMEM((2, R, K), jnp.float32),
            pltpu.VMEM((2, R, N), jnp.float32),
            pltpu.VMEM((2, R, N), jnp.float32),
            pltpu.SemaphoreType.DMA,
            pltpu.SemaphoreType.DMA((2,)),
            pltpu.SemaphoreType.DMA((2,)),
            pltpu.SemaphoreType.DMA((N_CHUNK,)),
        ],
        compiler_params=pltpu.CompilerParams(
            collective_id=0, vmem_limit_bytes=63 * 1024 * 1024
        ),
    )(O2, Wo)
